# Initial kernel scaffold; baseline (speedup 1.0000x reference)
import functools

import jax
import jax.numpy as jnp
from jax import lax
from jax.experimental import pallas as pl
from jax.experimental.pallas import tpu as pltpu
from jax.experimental.pallas import tpu_sc as plsc

NUM_ROWS = 17
DIM = 10

NC, NS = 2, 16
NW = NC * NS
N = 16384 * 200
CHUNK = N // NW
BLK = 2048
SUB = 128
NSUB = BLK // SUB
NBLK = CHUNK // BLK

_MESH = plsc.VectorSubcoreMesh(core_axis_name="c", subcore_axis_name="s")


@functools.partial(
    pl.kernel,
    out_type=jax.ShapeDtypeStruct((N, DIM), jnp.float32),
    mesh=_MESH,
    scratch_types=[
        pltpu.VMEM((BLK,), jnp.int32),
        pltpu.VMEM((BLK, DIM), jnp.float32),
        pltpu.SemaphoreType.DMA,
    ],
)
def _gather_kernel(idx_hbm, tab_hbm, out_hbm, idx_v, rows_v, sem):
    wid = lax.axis_index("s") * NC + lax.axis_index("c")
    base0 = wid * CHUNK

    @pl.loop(0, NBLK)
    def _(b):
        base = base0 + b * BLK
        pltpu.sync_copy(idx_hbm.at[pl.ds(base, BLK)], idx_v)
        copies = []
        for j in range(NSUB):
            sl = pl.ds(j * SUB, SUB)
            copies.append(
                pltpu.async_copy(tab_hbm.at[idx_v.at[sl]], rows_v.at[sl], sem)
            )
        for c in copies:
            c.wait()
        pltpu.sync_copy(rows_v, out_hbm.at[pl.ds(base, BLK)])


def kernel(list_POSs, table):
    idx_flat = list_POSs.reshape(-1).astype(jnp.int32)
    out = _gather_kernel(idx_flat, table)
    return out.reshape(list_POSs.shape + (DIM,))


# speedup vs baseline: 4.3915x; 4.3915x over previous
import dataclasses
import functools

import jax
import jax.numpy as jnp
from jax import lax
from jax.experimental import pallas as pl
from jax.experimental.pallas import tpu as pltpu
from jax.experimental.pallas import tpu_sc as plsc

NUM_ROWS = 17
DIM = 10
ROW_PAD = 24

NC, NS, L = 2, 16, 16
NW = NC * NS
N = 16384 * 200
CHUNK = N // NW
BLK = 2048
NGRP = BLK // L
NBLK = CHUNK // BLK

_MESH = plsc.VectorSubcoreMesh(core_axis_name="c", subcore_axis_name="s")

_CP = pltpu.CompilerParams()
if "needs_layout_passes" in pltpu.CompilerParams.__dataclass_fields__:
    _CP = dataclasses.replace(_CP, needs_layout_passes=False)


@functools.partial(
    pl.kernel,
    out_type=jax.ShapeDtypeStruct((N * DIM,), jnp.float32),
    mesh=_MESH,
    compiler_params=_CP,
    scratch_types=[
        pltpu.VMEM((DIM * ROW_PAD,), jnp.float32),
        pltpu.VMEM((BLK,), jnp.int32),
        pltpu.VMEM((BLK * DIM,), jnp.float32),
    ],
)
def _lookup_kernel(idx_hbm, tab_hbm, out_hbm, tab_v, idx_v, out_v):
    wid = lax.axis_index("s") * NC + lax.axis_index("c")
    base0 = wid * CHUNK
    pltpu.sync_copy(tab_hbm, tab_v)
    posbase = lax.iota(jnp.int32, L) * DIM

    @pl.loop(0, NBLK)
    def _(b):
        base = base0 + b * BLK
        pltpu.sync_copy(idx_hbm.at[pl.ds(base, BLK)], idx_v)

        @pl.loop(0, NGRP)
        def _(g):
            idxv = idx_v[pl.ds(g * L, L)]
            gbase = posbase + g * (L * DIM)
            for d in range(DIM):
                vals = plsc.load_gather(tab_v.at[pl.ds(d * ROW_PAD, ROW_PAD)],
                                        [idxv])
                plsc.store_scatter(out_v, [gbase + d], vals)

        pltpu.sync_copy(out_v, out_hbm.at[pl.ds(base * DIM, BLK * DIM)])


def kernel(list_POSs, table):
    idx_flat = list_POSs.reshape(-1).astype(jnp.int32)
    tab_dm = jnp.pad(table.astype(jnp.float32).T, ((0, 0), (0, ROW_PAD - NUM_ROWS)))
    out = _lookup_kernel(idx_flat, tab_dm.reshape(-1))
    return out.reshape(list_POSs.shape + (DIM,))


# speedup vs baseline: 93.5103x; 21.2936x over previous
import functools

import jax
import jax.numpy as jnp
from jax.experimental import pallas as pl
from jax.experimental.pallas import tpu as pltpu

NUM_ROWS = 17
DIM = 10
B = 16384
LEN = 200

BLK_B = 1024
GRID = B // BLK_B


def _lookup_body(tab_ref, idx_ref, out_ref):
    idxb = idx_ref[...]
    masks = [idxb == k for k in range(NUM_ROWS)]
    for d in range(DIM):
        acc = jnp.full(idxb.shape, tab_ref[NUM_ROWS - 1, d], dtype=jnp.float32)
        for k in range(NUM_ROWS - 1):
            acc = jnp.where(masks[k], tab_ref[k, d], acc)
        out_ref[d, :, :] = acc


@jax.jit
def _lookup(idx_t, table):
    return pl.pallas_call(
        _lookup_body,
        out_shape=jax.ShapeDtypeStruct((DIM, LEN, B), jnp.float32),
        grid=(GRID,),
        in_specs=[
            pl.BlockSpec(memory_space=pltpu.SMEM),
            pl.BlockSpec((LEN, BLK_B), lambda i: (0, i)),
        ],
        out_specs=pl.BlockSpec((DIM, LEN, BLK_B), lambda i: (0, 0, i)),
    )(table, idx_t)


def kernel(list_POSs, table):
    idx_t = list_POSs.astype(jnp.int32).T
    out_t = _lookup(idx_t, table.astype(jnp.float32))
    return jnp.transpose(out_t, (2, 1, 0))


# speedup vs baseline: 135.0016x; 1.4437x over previous
import jax
import jax.numpy as jnp
from jax.experimental import pallas as pl
from jax.experimental.pallas import tpu as pltpu

NUM_ROWS = 17
DIM = 10
B = 16384
LEN = 200

BLK_B = 1024
GRID = B // BLK_B


def _lookup_body(tab_ref, idx_ref, out_ref):
    idxb = idx_ref[...]
    for d in range(DIM):
        out_ref[d, :, :] = jnp.take_along_axis(
            tab_ref[d], idxb, axis=1, mode="promise_in_bounds"
        )


@jax.jit
def _lookup(idx_t, tab_lanes):
    return pl.pallas_call(
        _lookup_body,
        out_shape=jax.ShapeDtypeStruct((DIM, LEN, B), jnp.float32),
        grid=(GRID,),
        in_specs=[
            pl.BlockSpec((DIM, LEN, 128), lambda i: (0, 0, 0)),
            pl.BlockSpec((LEN, BLK_B), lambda i: (0, i)),
        ],
        out_specs=pl.BlockSpec((DIM, LEN, BLK_B), lambda i: (0, 0, i)),
    )(tab_lanes, idx_t)


def kernel(list_POSs, table):
    idx_t = list_POSs.astype(jnp.int32).T
    tab_lanes = jnp.broadcast_to(
        jnp.pad(table.astype(jnp.float32), ((0, 128 - NUM_ROWS), (0, 0))).T[:, None, :],
        (DIM, LEN, 128),
    )
    out_t = _lookup(idx_t, tab_lanes)
    return jnp.transpose(out_t, (2, 1, 0))
